# Initial kernel scaffold; baseline (speedup 1.0000x reference)
import jax
import jax.numpy as jnp
from jax.experimental import pallas as pl


def kernel(x, edge_index, batch, W1, b1, W2, b2, Wf1, bf1, Wf2, bf2):
    raise NotImplementedError("write your pallas kernel here")


# speedup vs baseline: 15.2800x; 15.2800x over previous
import functools

import jax
import jax.numpy as jnp
from jax import lax
from jax.experimental import pallas as pl
from jax.experimental.pallas import tpu as pltpu
from jax.experimental.pallas import tpu_sc as plsc

N = 10000
E = 320000
F_IN = 128
H = 64
C = 10
G = 128

NC = 2
NS = 16
NW = NC * NS
GRP = 128
GPT = 80
E_PAD = NW * GRP * GPT
N_PAD = 10112

_MESH = plsc.VectorSubcoreMesh(core_axis_name="c", subcore_axis_name="s")


_SC_PARAMS = pltpu.CompilerParams(use_tc_tiling_on_sc=False)


@functools.partial(
    pl.kernel,
    mesh=_MESH,
    compiler_params=_SC_PARAMS,
    out_type=jax.ShapeDtypeStruct((NC, N_PAD, 1), jnp.float32),
    scratch_types=[
        pltpu.VMEM((GPT, GRP), jnp.int32),
        pltpu.VMEM((GRP, 1), jnp.float32),
        pltpu.VMEM_SHARED((N_PAD, 1), jnp.float32),
    ],
)
def _deg_sc(dst_hbm, ones_hbm, zeros_hbm, out_hbm, dstv, onesv, acc):
    c = lax.axis_index("c")
    s = lax.axis_index("s")
    wid = s * NC + c
    zrows = N_PAD // NS
    pltpu.sync_copy(zeros_hbm.at[pl.ds(s * zrows, zrows)],
                    acc.at[pl.ds(s * zrows, zrows)])
    pltpu.sync_copy(dst_hbm.at[pl.ds(wid * GPT, GPT)], dstv)
    pltpu.sync_copy(ones_hbm, onesv)
    plsc.subcore_barrier()

    def body(j, carry):
        pltpu.sync_copy(onesv, acc.at[dstv.at[j]], add=True)
        return carry

    lax.fori_loop(0, GPT, body, 0)
    plsc.subcore_barrier()
    pltpu.sync_copy(acc.at[pl.ds(s * zrows, zrows)],
                    out_hbm.at[c, pl.ds(s * zrows, zrows)])


@functools.partial(
    pl.kernel,
    mesh=_MESH,
    compiler_params=_SC_PARAMS,
    out_type=jax.ShapeDtypeStruct((NC, N_PAD, H), jnp.float32),
    scratch_types=[
        pltpu.VMEM((GPT, GRP), jnp.int32),
        pltpu.VMEM((GPT, GRP), jnp.int32),
        pltpu.VMEM((GRP, H), jnp.float32),
        pltpu.VMEM_SHARED((N_PAD, H), jnp.float32),
        pltpu.SemaphoreType.DMA,
    ],
)
def _agg_sc(src_hbm, dst_hbm, hs_hbm, zeros_hbm, out_hbm,
            srcv, dstv, rows, acc, sem):
    c = lax.axis_index("c")
    s = lax.axis_index("s")
    wid = s * NC + c
    zrows = N_PAD // NS
    pltpu.sync_copy(zeros_hbm.at[pl.ds(s * zrows, zrows)],
                    acc.at[pl.ds(s * zrows, zrows)])
    pltpu.sync_copy(src_hbm.at[pl.ds(wid * GPT, GPT)], srcv)
    pltpu.sync_copy(dst_hbm.at[pl.ds(wid * GPT, GPT)], dstv)
    plsc.subcore_barrier()

    def body(j, carry):
        pltpu.async_copy(hs_hbm.at[srcv.at[j]], rows, sem).wait()
        pltpu.sync_copy(rows, acc.at[dstv.at[j]], add=True)
        return carry

    lax.fori_loop(0, GPT, body, 0)
    plsc.subcore_barrier()
    pltpu.sync_copy(acc.at[pl.ds(s * zrows, zrows)],
                    out_hbm.at[c, pl.ds(s * zrows, zrows)])


_BLK = 1000


def _tc_scale_body(degp_ref, x_ref, w_ref, dinv_ref, hs_ref):
    deg = degp_ref[0] + degp_ref[1] + 1.0
    dinv = lax.rsqrt(deg)
    h = jnp.dot(x_ref[...], w_ref[...], preferred_element_type=jnp.float32)
    dinv_ref[...] = dinv
    hs_ref[...] = h * dinv


def _tc_scale(deg_parts, x, w1):
    return pl.pallas_call(
        _tc_scale_body,
        grid=(N // _BLK,),
        in_specs=[
            pl.BlockSpec((NC, _BLK, 1), lambda i: (0, i, 0)),
            pl.BlockSpec((_BLK, F_IN), lambda i: (i, 0)),
            pl.BlockSpec((F_IN, H), lambda i: (0, 0)),
        ],
        out_specs=[
            pl.BlockSpec((_BLK, 1), lambda i: (i, 0)),
            pl.BlockSpec((_BLK, H), lambda i: (i, 0)),
        ],
        out_shape=[
            jax.ShapeDtypeStruct((N, 1), jnp.float32),
            jax.ShapeDtypeStruct((N, H), jnp.float32),
        ],
    )(deg_parts, x, w1)


def _tc_mid_body(aggp_ref, hs_ref, dinv_ref, b1_ref, w2_ref, out_ref):
    dinv = dinv_ref[...]
    t = dinv * (aggp_ref[0] + aggp_ref[1] + hs_ref[...]) + b1_ref[...]
    t = jnp.maximum(t, 0.0)
    h2 = jnp.dot(t, w2_ref[...], preferred_element_type=jnp.float32)
    out_ref[...] = h2 * dinv


def _tc_mid(agg_parts, hs1, dinv, b1, w2):
    return pl.pallas_call(
        _tc_mid_body,
        grid=(N // _BLK,),
        in_specs=[
            pl.BlockSpec((NC, _BLK, H), lambda i: (0, i, 0)),
            pl.BlockSpec((_BLK, H), lambda i: (i, 0)),
            pl.BlockSpec((_BLK, 1), lambda i: (i, 0)),
            pl.BlockSpec((1, H), lambda i: (0, 0)),
            pl.BlockSpec((H, H), lambda i: (0, 0)),
        ],
        out_specs=pl.BlockSpec((_BLK, H), lambda i: (i, 0)),
        out_shape=jax.ShapeDtypeStruct((N, H), jnp.float32),
    )(agg_parts, hs1, dinv, b1, w2)


def _tc_head_body(aggp_ref, hs_ref, dinv_ref, b2_ref, batch_ref,
                  wf1_ref, bf1_ref, wf2_ref, bf2_ref, out_ref):
    h3 = dinv_ref[...] * (aggp_ref[0] + aggp_ref[1] + hs_ref[...]) + b2_ref[...]
    h3 = jnp.maximum(h3, 0.0)
    gid = lax.broadcasted_iota(jnp.int32, (G, N), 0)
    oh = jnp.where(gid == batch_ref[...], 1.0, 0.0)
    sums = jnp.dot(oh, h3, preferred_element_type=jnp.float32)
    cnt = jnp.sum(oh, axis=1, keepdims=True)
    pooled = sums / jnp.maximum(cnt, 1.0)
    t = jnp.dot(pooled, wf1_ref[...], preferred_element_type=jnp.float32)
    t = jnp.maximum(t + bf1_ref[...], 0.0)
    logits = jnp.dot(t, wf2_ref[...], preferred_element_type=jnp.float32)
    logits = logits + bf2_ref[...]
    m = jnp.max(logits, axis=1, keepdims=True)
    lse = m + jnp.log(jnp.sum(jnp.exp(logits - m), axis=1, keepdims=True))
    out_ref[...] = logits - lse


def _tc_head(agg_parts, hs2, dinv, b2, batch_row, wf1, bf1, wf2, bf2):
    return pl.pallas_call(
        _tc_head_body,
        grid=(1,),
        in_specs=[
            pl.BlockSpec((NC, N, H), lambda i: (0, 0, 0)),
            pl.BlockSpec((N, H), lambda i: (0, 0)),
            pl.BlockSpec((N, 1), lambda i: (0, 0)),
            pl.BlockSpec((1, H), lambda i: (0, 0)),
            pl.BlockSpec((1, N), lambda i: (0, 0)),
            pl.BlockSpec((H, H), lambda i: (0, 0)),
            pl.BlockSpec((1, H), lambda i: (0, 0)),
            pl.BlockSpec((H, C), lambda i: (0, 0)),
            pl.BlockSpec((1, C), lambda i: (0, 0)),
        ],
        out_specs=pl.BlockSpec((G, C), lambda i: (0, 0)),
        out_shape=jax.ShapeDtypeStruct((G, C), jnp.float32),
    )(agg_parts, hs2, dinv, b2, batch_row, wf1, bf1, wf2, bf2)


def kernel(x, edge_index, batch, W1, b1, W2, b2, Wf1, bf1, Wf2, bf2):
    src = edge_index[0]
    dst = edge_index[1]
    pad = E_PAD - E
    src_p = jnp.concatenate(
        [src, jnp.zeros((pad,), jnp.int32)]).reshape(NW * GPT, GRP)
    dst_p = jnp.concatenate(
        [dst, jnp.full((pad,), N, jnp.int32)]).reshape(NW * GPT, GRP)
    zeros_h = jnp.zeros((N_PAD, H), jnp.float32)
    zeros_1 = jnp.zeros((N_PAD, 1), jnp.float32)
    ones_g = jnp.ones((GRP, 1), jnp.float32)

    deg_parts = _deg_sc(dst_p, ones_g, zeros_1)
    dinv, hs1 = _tc_scale(deg_parts[:, :N], x, W1)
    agg1 = _agg_sc(src_p, dst_p, hs1, zeros_h)
    hs2 = _tc_mid(agg1[:, :N], hs1, dinv, b1.reshape(1, H), W2)
    agg2 = _agg_sc(src_p, dst_p, hs2, zeros_h)
    return _tc_head(agg2[:, :N], hs2, dinv, b2.reshape(1, H),
                    batch.reshape(1, N), Wf1, bf1.reshape(1, H),
                    Wf2, bf2.reshape(1, C))


# speedup vs baseline: 16.7161x; 1.0940x over previous
import functools

import jax
import jax.numpy as jnp
from jax import lax
from jax.experimental import pallas as pl
from jax.experimental.pallas import tpu as pltpu
from jax.experimental.pallas import tpu_sc as plsc

N = 10000
E = 320000
F_IN = 128
H = 64
C = 10
G = 128

NC = 2
NS = 16
NW = NC * NS
GRP = 128
GPT = 80
E_PAD = NW * GRP * GPT
N_PAD = 10112

_MESH = plsc.VectorSubcoreMesh(core_axis_name="c", subcore_axis_name="s")


_SC_PARAMS = pltpu.CompilerParams(use_tc_tiling_on_sc=False)


@functools.partial(
    pl.kernel,
    mesh=_MESH,
    compiler_params=_SC_PARAMS,
    out_type=jax.ShapeDtypeStruct((NC, N_PAD, 1), jnp.float32),
    scratch_types=[
        pltpu.VMEM((GPT, GRP), jnp.int32),
        pltpu.VMEM((GRP, 1), jnp.float32),
        pltpu.VMEM_SHARED((N_PAD, 1), jnp.float32),
    ],
)
def _deg_sc(dst_hbm, ones_hbm, zeros_hbm, out_hbm, dstv, onesv, acc):
    c = lax.axis_index("c")
    s = lax.axis_index("s")
    wid = s * NC + c
    zrows = N_PAD // NS
    pltpu.sync_copy(zeros_hbm.at[pl.ds(s * zrows, zrows)],
                    acc.at[pl.ds(s * zrows, zrows)])
    pltpu.sync_copy(dst_hbm.at[pl.ds(wid * GPT, GPT)], dstv)
    pltpu.sync_copy(ones_hbm, onesv)
    plsc.subcore_barrier()

    def body(j, carry):
        pltpu.sync_copy(onesv, acc.at[dstv.at[j]], add=True)
        return carry

    lax.fori_loop(0, GPT, body, 0)
    plsc.subcore_barrier()
    pltpu.sync_copy(acc.at[pl.ds(s * zrows, zrows)],
                    out_hbm.at[c, pl.ds(s * zrows, zrows)])


@functools.partial(
    pl.kernel,
    mesh=_MESH,
    compiler_params=_SC_PARAMS,
    out_type=jax.ShapeDtypeStruct((NC, N_PAD, H), jnp.float32),
    scratch_types=[
        pltpu.VMEM((GPT, GRP), jnp.int32),
        pltpu.VMEM((GPT, GRP), jnp.int32),
        pltpu.VMEM((2, GRP, H), jnp.float32),
        pltpu.VMEM_SHARED((N_PAD, H), jnp.float32),
        pltpu.SemaphoreType.DMA((2,)),
        pltpu.SemaphoreType.DMA((2,)),
    ],
)
def _agg_sc(src_hbm, dst_hbm, hs_hbm, zeros_hbm, out_hbm,
            srcv, dstv, rows, acc, semg, sems):
    c = lax.axis_index("c")
    s = lax.axis_index("s")
    wid = s * NC + c
    zrows = N_PAD // NS
    pltpu.sync_copy(zeros_hbm.at[pl.ds(s * zrows, zrows)],
                    acc.at[pl.ds(s * zrows, zrows)])
    pltpu.sync_copy(src_hbm.at[pl.ds(wid * GPT, GPT)], srcv)
    pltpu.sync_copy(dst_hbm.at[pl.ds(wid * GPT, GPT)], dstv)
    plsc.subcore_barrier()
    pltpu.async_copy(hs_hbm.at[srcv.at[0]], rows.at[0], semg.at[0])

    def body(j, carry):
        b = lax.rem(j, 2)
        nb = 1 - b
        pltpu.make_async_copy(hs_hbm.at[srcv.at[j]], rows.at[b],
                              semg.at[b]).wait()
        pltpu.async_copy(rows.at[b], acc.at[dstv.at[j]], sems.at[b], add=True)

        @pl.when(j >= 1)
        def _():
            pltpu.make_async_copy(rows.at[nb], acc.at[dstv.at[j - 1]],
                                  sems.at[nb]).wait()

        @pl.when(j < GPT - 1)
        def _():
            pltpu.async_copy(hs_hbm.at[srcv.at[j + 1]], rows.at[nb],
                             semg.at[nb])

        return carry

    lax.fori_loop(0, GPT, body, 0)
    lb = (GPT - 1) % 2
    pltpu.make_async_copy(rows.at[lb], acc.at[dstv.at[GPT - 1]],
                          sems.at[lb]).wait()
    plsc.subcore_barrier()
    pltpu.sync_copy(acc.at[pl.ds(s * zrows, zrows)],
                    out_hbm.at[c, pl.ds(s * zrows, zrows)])


_BLK = 1000


def _tc_scale_body(degp_ref, x_ref, w_ref, dinv_ref, hs_ref):
    deg = degp_ref[0] + degp_ref[1] + 1.0
    dinv = lax.rsqrt(deg)
    h = jnp.dot(x_ref[...], w_ref[...], preferred_element_type=jnp.float32)
    dinv_ref[...] = dinv
    hs_ref[...] = h * dinv


def _tc_scale(deg_parts, x, w1):
    return pl.pallas_call(
        _tc_scale_body,
        grid=(N // _BLK,),
        in_specs=[
            pl.BlockSpec((NC, _BLK, 1), lambda i: (0, i, 0)),
            pl.BlockSpec((_BLK, F_IN), lambda i: (i, 0)),
            pl.BlockSpec((F_IN, H), lambda i: (0, 0)),
        ],
        out_specs=[
            pl.BlockSpec((_BLK, 1), lambda i: (i, 0)),
            pl.BlockSpec((_BLK, H), lambda i: (i, 0)),
        ],
        out_shape=[
            jax.ShapeDtypeStruct((N, 1), jnp.float32),
            jax.ShapeDtypeStruct((N, H), jnp.float32),
        ],
    )(deg_parts, x, w1)


def _tc_mid_body(aggp_ref, hs_ref, dinv_ref, b1_ref, w2_ref, out_ref):
    dinv = dinv_ref[...]
    t = dinv * (aggp_ref[0] + aggp_ref[1] + hs_ref[...]) + b1_ref[...]
    t = jnp.maximum(t, 0.0)
    h2 = jnp.dot(t, w2_ref[...], preferred_element_type=jnp.float32)
    out_ref[...] = h2 * dinv


def _tc_mid(agg_parts, hs1, dinv, b1, w2):
    return pl.pallas_call(
        _tc_mid_body,
        grid=(N // _BLK,),
        in_specs=[
            pl.BlockSpec((NC, _BLK, H), lambda i: (0, i, 0)),
            pl.BlockSpec((_BLK, H), lambda i: (i, 0)),
            pl.BlockSpec((_BLK, 1), lambda i: (i, 0)),
            pl.BlockSpec((1, H), lambda i: (0, 0)),
            pl.BlockSpec((H, H), lambda i: (0, 0)),
        ],
        out_specs=pl.BlockSpec((_BLK, H), lambda i: (i, 0)),
        out_shape=jax.ShapeDtypeStruct((N, H), jnp.float32),
    )(agg_parts, hs1, dinv, b1, w2)


def _tc_head_body(aggp_ref, hs_ref, dinv_ref, b2_ref, batch_ref,
                  wf1_ref, bf1_ref, wf2_ref, bf2_ref, out_ref):
    h3 = dinv_ref[...] * (aggp_ref[0] + aggp_ref[1] + hs_ref[...]) + b2_ref[...]
    h3 = jnp.maximum(h3, 0.0)
    gid = lax.broadcasted_iota(jnp.int32, (G, N), 0)
    oh = jnp.where(gid == batch_ref[...], 1.0, 0.0)
    sums = jnp.dot(oh, h3, preferred_element_type=jnp.float32)
    cnt = jnp.sum(oh, axis=1, keepdims=True)
    pooled = sums / jnp.maximum(cnt, 1.0)
    t = jnp.dot(pooled, wf1_ref[...], preferred_element_type=jnp.float32)
    t = jnp.maximum(t + bf1_ref[...], 0.0)
    logits = jnp.dot(t, wf2_ref[...], preferred_element_type=jnp.float32)
    logits = logits + bf2_ref[...]
    m = jnp.max(logits, axis=1, keepdims=True)
    lse = m + jnp.log(jnp.sum(jnp.exp(logits - m), axis=1, keepdims=True))
    out_ref[...] = logits - lse


def _tc_head(agg_parts, hs2, dinv, b2, batch_row, wf1, bf1, wf2, bf2):
    return pl.pallas_call(
        _tc_head_body,
        grid=(1,),
        in_specs=[
            pl.BlockSpec((NC, N, H), lambda i: (0, 0, 0)),
            pl.BlockSpec((N, H), lambda i: (0, 0)),
            pl.BlockSpec((N, 1), lambda i: (0, 0)),
            pl.BlockSpec((1, H), lambda i: (0, 0)),
            pl.BlockSpec((1, N), lambda i: (0, 0)),
            pl.BlockSpec((H, H), lambda i: (0, 0)),
            pl.BlockSpec((1, H), lambda i: (0, 0)),
            pl.BlockSpec((H, C), lambda i: (0, 0)),
            pl.BlockSpec((1, C), lambda i: (0, 0)),
        ],
        out_specs=pl.BlockSpec((G, C), lambda i: (0, 0)),
        out_shape=jax.ShapeDtypeStruct((G, C), jnp.float32),
    )(agg_parts, hs2, dinv, b2, batch_row, wf1, bf1, wf2, bf2)


def kernel(x, edge_index, batch, W1, b1, W2, b2, Wf1, bf1, Wf2, bf2):
    src = edge_index[0]
    dst = edge_index[1]
    pad = E_PAD - E
    src_p = jnp.concatenate(
        [src, jnp.zeros((pad,), jnp.int32)]).reshape(NW * GPT, GRP)
    dst_p = jnp.concatenate(
        [dst, jnp.full((pad,), N, jnp.int32)]).reshape(NW * GPT, GRP)
    zeros_h = jnp.zeros((N_PAD, H), jnp.float32)
    zeros_1 = jnp.zeros((N_PAD, 1), jnp.float32)
    ones_g = jnp.ones((GRP, 1), jnp.float32)

    deg_parts = _deg_sc(dst_p, ones_g, zeros_1)
    dinv, hs1 = _tc_scale(deg_parts, x, W1)
    agg1 = _agg_sc(src_p, dst_p, hs1, zeros_h)
    hs2 = _tc_mid(agg1, hs1, dinv, b1.reshape(1, H), W2)
    agg2 = _agg_sc(src_p, dst_p, hs2, zeros_h)
    return _tc_head(agg2, hs2, dinv, b2.reshape(1, H),
                    batch.reshape(1, N), Wf1, bf1.reshape(1, H),
                    Wf2, bf2.reshape(1, C))


# speedup vs baseline: 18.0884x; 1.0821x over previous
import functools

import jax
import jax.numpy as jnp
from jax import lax
from jax.experimental import pallas as pl
from jax.experimental.pallas import tpu as pltpu
from jax.experimental.pallas import tpu_sc as plsc

N = 10000
E = 320000
F_IN = 128
H = 64
C = 10
G = 128

NC = 2
NS = 16
NW = NC * NS
GRP = 128
GPT = 80
E_PAD = NW * GRP * GPT
N_PAD = 10112

_MESH = plsc.VectorSubcoreMesh(core_axis_name="c", subcore_axis_name="s")


_SC_PARAMS = pltpu.CompilerParams(use_tc_tiling_on_sc=False)


@functools.partial(
    pl.kernel,
    mesh=_MESH,
    compiler_params=_SC_PARAMS,
    out_type=jax.ShapeDtypeStruct((NC, N_PAD, 1), jnp.float32),
    scratch_types=[
        pltpu.VMEM((GPT, GRP), jnp.int32),
        pltpu.VMEM((GRP, 1), jnp.float32),
        pltpu.VMEM_SHARED((N_PAD, 1), jnp.float32),
    ],
)
def _deg_sc(dst_hbm, ones_hbm, zeros_hbm, out_hbm, dstv, onesv, acc):
    c = lax.axis_index("c")
    s = lax.axis_index("s")
    wid = s * NC + c
    zrows = N_PAD // NS
    pltpu.sync_copy(zeros_hbm.at[pl.ds(s * zrows, zrows)],
                    acc.at[pl.ds(s * zrows, zrows)])
    pltpu.sync_copy(dst_hbm.at[pl.ds(wid * GPT, GPT)], dstv)
    pltpu.sync_copy(ones_hbm, onesv)
    plsc.subcore_barrier()

    def body(j, carry):
        pltpu.sync_copy(onesv, acc.at[dstv.at[j]], add=True)
        return carry

    lax.fori_loop(0, GPT, body, 0)
    plsc.subcore_barrier()
    pltpu.sync_copy(acc.at[pl.ds(s * zrows, zrows)],
                    out_hbm.at[c, pl.ds(s * zrows, zrows)])


@functools.partial(
    pl.kernel,
    mesh=_MESH,
    compiler_params=_SC_PARAMS,
    out_type=jax.ShapeDtypeStruct((NC, N_PAD, H), jnp.float32),
    scratch_types=[
        pltpu.VMEM((GPT, GRP), jnp.int32),
        pltpu.VMEM((GPT, GRP), jnp.int32),
        pltpu.VMEM((GRP, H), jnp.float32),
        pltpu.VMEM((GRP, H), jnp.float32),
        pltpu.VMEM_SHARED((N_PAD, H), jnp.float32),
        pltpu.SemaphoreType.DMA,
    ],
)
def _agg_sc(src_hbm, dst_hbm, hs_hbm, zeros_hbm, out_hbm,
            srcv, dstv, rows0, rows1, acc, semg):
    c = lax.axis_index("c")
    s = lax.axis_index("s")
    wid = s * NC + c
    zrows = N_PAD // NS
    pltpu.sync_copy(zeros_hbm.at[pl.ds(s * zrows, zrows)],
                    acc.at[pl.ds(s * zrows, zrows)])
    pltpu.sync_copy(src_hbm.at[pl.ds(wid * GPT, GPT)], srcv)
    pltpu.sync_copy(dst_hbm.at[pl.ds(wid * GPT, GPT)], dstv)
    plsc.subcore_barrier()
    pltpu.async_copy(hs_hbm.at[srcv.at[0]], rows0, semg)

    def body(j, carry):
        even = lax.rem(j, 2) == 0

        @pl.when(j < GPT - 1)
        def _():
            @pl.when(even)
            def _():
                pltpu.async_copy(hs_hbm.at[srcv.at[j + 1]], rows1, semg)

            @pl.when(jnp.logical_not(even))
            def _():
                pltpu.async_copy(hs_hbm.at[srcv.at[j + 1]], rows0, semg)

        @pl.when(even)
        def _():
            pltpu.make_async_copy(hs_hbm.at[srcv.at[j]], rows0, semg).wait()
            pltpu.sync_copy(rows0, acc.at[dstv.at[j]], add=True)

        @pl.when(jnp.logical_not(even))
        def _():
            pltpu.make_async_copy(hs_hbm.at[srcv.at[j]], rows1, semg).wait()
            pltpu.sync_copy(rows1, acc.at[dstv.at[j]], add=True)

        return carry

    lax.fori_loop(0, GPT, body, 0)
    plsc.subcore_barrier()
    pltpu.sync_copy(acc.at[pl.ds(s * zrows, zrows)],
                    out_hbm.at[c, pl.ds(s * zrows, zrows)])


_BLK = 1000


def _tc_scale_body(degp_ref, x_ref, w_ref, dinv_ref, hs_ref):
    deg = degp_ref[0] + degp_ref[1] + 1.0
    dinv = lax.rsqrt(deg)
    h = jnp.dot(x_ref[...], w_ref[...], preferred_element_type=jnp.float32)
    dinv_ref[...] = dinv
    hs_ref[...] = h * dinv


def _tc_scale(deg_parts, x, w1):
    return pl.pallas_call(
        _tc_scale_body,
        grid=(N // _BLK,),
        in_specs=[
            pl.BlockSpec((NC, _BLK, 1), lambda i: (0, i, 0)),
            pl.BlockSpec((_BLK, F_IN), lambda i: (i, 0)),
            pl.BlockSpec((F_IN, H), lambda i: (0, 0)),
        ],
        out_specs=[
            pl.BlockSpec((_BLK, 1), lambda i: (i, 0)),
            pl.BlockSpec((_BLK, H), lambda i: (i, 0)),
        ],
        out_shape=[
            jax.ShapeDtypeStruct((N, 1), jnp.float32),
            jax.ShapeDtypeStruct((N, H), jnp.float32),
        ],
    )(deg_parts, x, w1)


def _tc_mid_body(aggp_ref, hs_ref, dinv_ref, b1_ref, w2_ref, out_ref):
    dinv = dinv_ref[...]
    t = dinv * (aggp_ref[0] + aggp_ref[1] + hs_ref[...]) + b1_ref[...]
    t = jnp.maximum(t, 0.0)
    h2 = jnp.dot(t, w2_ref[...], preferred_element_type=jnp.float32)
    out_ref[...] = h2 * dinv


def _tc_mid(agg_parts, hs1, dinv, b1, w2):
    return pl.pallas_call(
        _tc_mid_body,
        grid=(N // _BLK,),
        in_specs=[
            pl.BlockSpec((NC, _BLK, H), lambda i: (0, i, 0)),
            pl.BlockSpec((_BLK, H), lambda i: (i, 0)),
            pl.BlockSpec((_BLK, 1), lambda i: (i, 0)),
            pl.BlockSpec((1, H), lambda i: (0, 0)),
            pl.BlockSpec((H, H), lambda i: (0, 0)),
        ],
        out_specs=pl.BlockSpec((_BLK, H), lambda i: (i, 0)),
        out_shape=jax.ShapeDtypeStruct((N, H), jnp.float32),
    )(agg_parts, hs1, dinv, b1, w2)


def _tc_head_body(aggp_ref, hs_ref, dinv_ref, b2_ref, batch_ref,
                  wf1_ref, bf1_ref, wf2_ref, bf2_ref, out_ref):
    h3 = dinv_ref[...] * (aggp_ref[0] + aggp_ref[1] + hs_ref[...]) + b2_ref[...]
    h3 = jnp.maximum(h3, 0.0)
    gid = lax.broadcasted_iota(jnp.int32, (G, N), 0)
    oh = jnp.where(gid == batch_ref[...], 1.0, 0.0)
    sums = jnp.dot(oh, h3, preferred_element_type=jnp.float32)
    cnt = jnp.sum(oh, axis=1, keepdims=True)
    pooled = sums / jnp.maximum(cnt, 1.0)
    t = jnp.dot(pooled, wf1_ref[...], preferred_element_type=jnp.float32)
    t = jnp.maximum(t + bf1_ref[...], 0.0)
    logits = jnp.dot(t, wf2_ref[...], preferred_element_type=jnp.float32)
    logits = logits + bf2_ref[...]
    m = jnp.max(logits, axis=1, keepdims=True)
    lse = m + jnp.log(jnp.sum(jnp.exp(logits - m), axis=1, keepdims=True))
    out_ref[...] = logits - lse


def _tc_head(agg_parts, hs2, dinv, b2, batch_row, wf1, bf1, wf2, bf2):
    return pl.pallas_call(
        _tc_head_body,
        grid=(1,),
        in_specs=[
            pl.BlockSpec((NC, N, H), lambda i: (0, 0, 0)),
            pl.BlockSpec((N, H), lambda i: (0, 0)),
            pl.BlockSpec((N, 1), lambda i: (0, 0)),
            pl.BlockSpec((1, H), lambda i: (0, 0)),
            pl.BlockSpec((1, N), lambda i: (0, 0)),
            pl.BlockSpec((H, H), lambda i: (0, 0)),
            pl.BlockSpec((1, H), lambda i: (0, 0)),
            pl.BlockSpec((H, C), lambda i: (0, 0)),
            pl.BlockSpec((1, C), lambda i: (0, 0)),
        ],
        out_specs=pl.BlockSpec((G, C), lambda i: (0, 0)),
        out_shape=jax.ShapeDtypeStruct((G, C), jnp.float32),
    )(agg_parts, hs2, dinv, b2, batch_row, wf1, bf1, wf2, bf2)


def kernel(x, edge_index, batch, W1, b1, W2, b2, Wf1, bf1, Wf2, bf2):
    src = edge_index[0]
    dst = edge_index[1]
    pad = E_PAD - E
    src_p = jnp.concatenate(
        [src, jnp.zeros((pad,), jnp.int32)]).reshape(NW * GPT, GRP)
    dst_p = jnp.concatenate(
        [dst, jnp.full((pad,), N, jnp.int32)]).reshape(NW * GPT, GRP)
    zeros_h = jnp.zeros((N_PAD, H), jnp.float32)
    zeros_1 = jnp.zeros((N_PAD, 1), jnp.float32)
    ones_g = jnp.ones((GRP, 1), jnp.float32)

    deg_parts = _deg_sc(dst_p, ones_g, zeros_1)
    dinv, hs1 = _tc_scale(deg_parts, x, W1)
    agg1 = _agg_sc(src_p, dst_p, hs1, zeros_h)
    hs2 = _tc_mid(agg1, hs1, dinv, b1.reshape(1, H), W2)
    agg2 = _agg_sc(src_p, dst_p, hs2, zeros_h)
    return _tc_head(agg2, hs2, dinv, b2.reshape(1, H),
                    batch.reshape(1, N), Wf1, bf1.reshape(1, H),
                    Wf2, bf2.reshape(1, C))
